# Initial kernel scaffold; baseline (speedup 1.0000x reference)
import jax
import jax.numpy as jnp
from jax.experimental import pallas as pl


def kernel(z_e_x, codebook):
    raise NotImplementedError("write your pallas kernel here")


# speedup vs baseline: 1.2645x; 1.2645x over previous
import jax
import jax.numpy as jnp
from jax.experimental import pallas as pl
from jax.experimental.pallas import tpu as pltpu

_K = 8192
_TN = 256


def _vq_argmin_kernel(f_ref, cbt_ref, out_ref):
    f = f_ref[...]
    cbt = cbt_ref[...]
    m = jnp.dot(f, cbt, preferred_element_type=jnp.float32)
    f2 = jnp.sum(f * f, axis=1, keepdims=True)
    c2 = jnp.sum(cbt * cbt, axis=0, keepdims=True)
    dist = (f2 - 2.0 * m) + c2
    mn = jnp.min(dist, axis=1, keepdims=True)
    kidx = jax.lax.broadcasted_iota(jnp.int32, dist.shape, 1)
    idx = jnp.min(jnp.where(dist == mn, kidx, _K), axis=1)
    out_ref[...] = idx.reshape(1, 1, _TN)


def kernel(z_e_x, codebook):
    B, D, H, W = z_e_x.shape
    flat = jnp.transpose(z_e_x, (0, 2, 3, 1)).reshape(-1, D)
    N = flat.shape[0]
    cbt = codebook.T
    grid = (N // _TN,)
    out = pl.pallas_call(
        _vq_argmin_kernel,
        grid=grid,
        in_specs=[
            pl.BlockSpec((_TN, D), lambda i: (i, 0)),
            pl.BlockSpec((D, _K), lambda i: (0, 0)),
        ],
        out_specs=pl.BlockSpec((1, 1, _TN), lambda i: (i, 0, 0)),
        out_shape=jax.ShapeDtypeStruct((N // _TN, 1, _TN), jnp.int32),
        compiler_params=pltpu.CompilerParams(
            dimension_semantics=("parallel",),
        ),
    )(flat, cbt)
    return out.reshape(B, H, W)


# speedup vs baseline: 1.3645x; 1.0792x over previous
import jax
import jax.numpy as jnp
from jax.experimental import pallas as pl
from jax.experimental.pallas import tpu as pltpu

_K = 8192
_TN = 256


def _vq_argmin_kernel(f_ref, cbt_ref, out_ref):
    f = f_ref[...]
    cbt = cbt_ref[...]
    m = jnp.dot(f, cbt, preferred_element_type=jnp.float32)
    f2 = jnp.sum(f * f, axis=1, keepdims=True)
    c2 = jnp.sum(cbt * cbt, axis=0, keepdims=True)
    dist = (f2 - 2.0 * m) + c2
    idx = jnp.argmin(dist, axis=1).astype(jnp.int32)
    out_ref[...] = idx.reshape(1, 1, _TN)


def kernel(z_e_x, codebook):
    B, D, H, W = z_e_x.shape
    flat = jnp.transpose(z_e_x, (0, 2, 3, 1)).reshape(-1, D)
    N = flat.shape[0]
    cbt = codebook.T
    grid = (N // _TN,)
    out = pl.pallas_call(
        _vq_argmin_kernel,
        grid=grid,
        in_specs=[
            pl.BlockSpec((_TN, D), lambda i: (i, 0)),
            pl.BlockSpec((D, _K), lambda i: (0, 0)),
        ],
        out_specs=pl.BlockSpec((1, 1, _TN), lambda i: (i, 0, 0)),
        out_shape=jax.ShapeDtypeStruct((N // _TN, 1, _TN), jnp.int32),
        compiler_params=pltpu.CompilerParams(
            dimension_semantics=("parallel",),
        ),
    )(flat, cbt)
    return out.reshape(B, H, W)


# speedup vs baseline: 1.5267x; 1.1189x over previous
import jax
import jax.numpy as jnp
from jax.experimental import pallas as pl
from jax.experimental.pallas import tpu as pltpu

_K = 8192
_TN = 512


def _vq_argmin_kernel(f_ref, cbt_ref, out_ref):
    f = f_ref[...]
    cbt = cbt_ref[...]
    m = jnp.dot(f, cbt, preferred_element_type=jnp.float32)
    f2 = jnp.sum(f * f, axis=1, keepdims=True)
    c2 = jnp.sum(cbt * cbt, axis=0, keepdims=True)
    dist = (f2 - 2.0 * m) + c2
    idx = jnp.argmin(dist, axis=1).astype(jnp.int32)
    out_ref[...] = idx.reshape(1, 1, _TN)


def kernel(z_e_x, codebook):
    B, D, H, W = z_e_x.shape
    flat = jnp.transpose(z_e_x, (0, 2, 3, 1)).reshape(-1, D)
    N = flat.shape[0]
    cbt = codebook.T
    grid = (N // _TN,)
    out = pl.pallas_call(
        _vq_argmin_kernel,
        grid=grid,
        in_specs=[
            pl.BlockSpec((_TN, D), lambda i: (i, 0)),
            pl.BlockSpec((D, _K), lambda i: (0, 0)),
        ],
        out_specs=pl.BlockSpec((1, 1, _TN), lambda i: (i, 0, 0)),
        out_shape=jax.ShapeDtypeStruct((N // _TN, 1, _TN), jnp.int32),
        compiler_params=pltpu.CompilerParams(
            dimension_semantics=("parallel",),
        ),
    )(flat, cbt)
    return out.reshape(B, H, W)


# speedup vs baseline: 1.6062x; 1.0520x over previous
import jax
import jax.numpy as jnp
from jax.experimental import pallas as pl
from jax.experimental.pallas import tpu as pltpu

_K = 8192
_TN = 1024


def _vq_argmin_kernel(f_ref, cbt_ref, out_ref):
    f = f_ref[...]
    cbt = cbt_ref[...]
    m = jnp.dot(f, cbt, preferred_element_type=jnp.float32)
    f2 = jnp.sum(f * f, axis=1, keepdims=True)
    c2 = jnp.sum(cbt * cbt, axis=0, keepdims=True)
    dist = (f2 - 2.0 * m) + c2
    idx = jnp.argmin(dist, axis=1).astype(jnp.int32)
    out_ref[...] = idx.reshape(1, 1, _TN)


def kernel(z_e_x, codebook):
    B, D, H, W = z_e_x.shape
    flat = jnp.transpose(z_e_x, (0, 2, 3, 1)).reshape(-1, D)
    N = flat.shape[0]
    cbt = codebook.T
    grid = (N // _TN,)
    out = pl.pallas_call(
        _vq_argmin_kernel,
        grid=grid,
        in_specs=[
            pl.BlockSpec((_TN, D), lambda i: (i, 0)),
            pl.BlockSpec((D, _K), lambda i: (0, 0)),
        ],
        out_specs=pl.BlockSpec((1, 1, _TN), lambda i: (i, 0, 0)),
        out_shape=jax.ShapeDtypeStruct((N // _TN, 1, _TN), jnp.int32),
        compiler_params=pltpu.CompilerParams(
            dimension_semantics=("parallel",),
        ),
    )(flat, cbt)
    return out.reshape(B, H, W)


# speedup vs baseline: 1.6309x; 1.0154x over previous
import jax
import jax.numpy as jnp
from jax.experimental import pallas as pl
from jax.experimental.pallas import tpu as pltpu

_K = 8192
_TN = 2048


def _vq_argmin_kernel(f_ref, cbt_ref, out_ref):
    f = f_ref[...]
    cbt = cbt_ref[...]
    m = jnp.dot(f, cbt, preferred_element_type=jnp.float32)
    f2 = jnp.sum(f * f, axis=1, keepdims=True)
    c2 = jnp.sum(cbt * cbt, axis=0, keepdims=True)
    dist = (f2 - 2.0 * m) + c2
    idx = jnp.argmin(dist, axis=1).astype(jnp.int32)
    out_ref[...] = idx.reshape(1, 1, _TN)


def kernel(z_e_x, codebook):
    B, D, H, W = z_e_x.shape
    flat = jnp.transpose(z_e_x, (0, 2, 3, 1)).reshape(-1, D)
    N = flat.shape[0]
    cbt = codebook.T
    grid = (N // _TN,)
    out = pl.pallas_call(
        _vq_argmin_kernel,
        grid=grid,
        in_specs=[
            pl.BlockSpec((_TN, D), lambda i: (i, 0)),
            pl.BlockSpec((D, _K), lambda i: (0, 0)),
        ],
        out_specs=pl.BlockSpec((1, 1, _TN), lambda i: (i, 0, 0)),
        out_shape=jax.ShapeDtypeStruct((N // _TN, 1, _TN), jnp.int32),
        compiler_params=pltpu.CompilerParams(
            dimension_semantics=("parallel",),
        ),
    )(flat, cbt)
    return out.reshape(B, H, W)
